# baseline (speedup 1.0000x reference)
import functools

import jax
import jax.numpy as jnp
from jax import lax
from jax.experimental import pallas as pl
from jax.experimental.pallas import tpu as pltpu
from jax.experimental.pallas import tpu_sc as plsc

_B = 16384
_D = 64
_NC = 2
_NS = 16
_L = 16
_NW = _NC * _NS
_RPW = _B // _NW
_NG = _RPW // _L
_NCHUNK = 4
_CHUNK = _RPW // _NCHUNK


def _sc_partials(users, items, scores, sample_weight, utb, itb):
    mesh = plsc.VectorSubcoreMesh(core_axis_name="c", subcore_axis_name="s")

    @functools.partial(
        pl.kernel,
        mesh=mesh,
        out_type=jax.ShapeDtypeStruct((_NW, _L), jnp.float32),
        compiler_params=pltpu.CompilerParams(
            needs_layout_passes=False, use_tc_tiling_on_sc=False),
        scratch_types=[
            pltpu.VMEM((_NCHUNK, _CHUNK), jnp.int32),
            pltpu.VMEM((_NCHUNK, _CHUNK), jnp.int32),
            pltpu.VMEM((_RPW,), jnp.float32),
            pltpu.VMEM((_RPW,), jnp.float32),
            pltpu.VMEM((_RPW, _D), jnp.bfloat16),
            pltpu.VMEM((_RPW, _D), jnp.bfloat16),
            pltpu.VMEM((_L,), jnp.float32),
            pltpu.SemaphoreType.DMA,
        ],
    )
    def k(users_h, items_h, scores_h, w_h, ut_h, it_h, out_h,
          uidx, iidx, sc_v, w_v, urows, irows, ostage, sem):
        wid = lax.axis_index("s") * _NC + lax.axis_index("c")
        base = wid * _RPW

        for j in range(_NCHUNK):
            off = base + j * _CHUNK
            pltpu.sync_copy(users_h.at[pl.ds(off, _CHUNK)], uidx.at[j])
            pltpu.sync_copy(items_h.at[pl.ds(off, _CHUNK)], iidx.at[j])
        pltpu.sync_copy(scores_h.at[pl.ds(base, _RPW)], sc_v)
        pltpu.sync_copy(w_h.at[pl.ds(base, _RPW)], w_v)

        copies = []
        for j in range(_NCHUNK):
            dst = pl.ds(j * _CHUNK, _CHUNK)
            copies.append(pltpu.async_copy(ut_h.at[uidx.at[j]], urows.at[dst], sem))
            copies.append(pltpu.async_copy(it_h.at[iidx.at[j]], irows.at[dst], sem))
        for c in copies:
            c.wait()

        def group(g, acc):
            r0 = g * _L
            svec = sc_v[pl.ds(r0, _L)]
            wvec = w_v[pl.ds(r0, _L)]
            for j in range(_L):
                r = r0 + j
                p = None
                for c in range(_D // (2 * _L)):
                    ub = urows[r, pl.ds(c * 2 * _L, 2 * _L)]
                    ib = irows[r, pl.ds(c * 2 * _L, 2 * _L)]
                    ua, ub2 = plsc.unpack(ub, format=plsc.PackFormat.INTERLEAVED)
                    ia, ib2 = plsc.unpack(ib, format=plsc.PackFormat.INTERLEAVED)
                    q = ua * ia + ub2 * ib2
                    p = q if p is None else p + q
                pred = jnp.sum(p)
                e = pred - svec[j]
                acc = acc + e * e * wvec[j]
            return acc

        total = lax.fori_loop(0, _NG, group, jnp.float32(0.0))
        ostage[...] = jnp.where(lax.iota(jnp.int32, _L) == 0, total, 0.0)
        pltpu.sync_copy(ostage, out_h.at[wid])

    return k(users, items, scores, sample_weight, utb, itb)


def _finalize(partials):
    def body(p_ref, o_ref):
        o_ref[0, 0] = jnp.sum(p_ref[...]) * (1.0 / _B)

    out = pl.pallas_call(
        body,
        out_shape=jax.ShapeDtypeStruct((1, 1), jnp.float32),
        out_specs=pl.BlockSpec(memory_space=pltpu.SMEM),
    )(partials)
    return out[0, 0]


def kernel(users, items, scores, sample_weight, user_table, item_table):
    partials = _sc_partials(users, items, scores, sample_weight,
                            user_table.astype(jnp.bfloat16),
                            item_table.astype(jnp.bfloat16))
    return _finalize(partials)


# speedup vs baseline: 1.3085x; 1.3085x over previous
import functools

import jax
import jax.numpy as jnp
from jax import lax
from jax.experimental import pallas as pl
from jax.experimental.pallas import tpu as pltpu
from jax.experimental.pallas import tpu_sc as plsc

_B = 16384
_D = 64
_NC = 2
_NS = 16
_L = 16
_NW = _NC * _NS
_RPW = _B // _NW
_NG = _RPW // _L
_NCHUNK = 4
_CHUNK = _RPW // _NCHUNK

_MESH = dict(core_axis_name="c", subcore_axis_name="s")
_PARAMS = pltpu.CompilerParams(
    needs_layout_passes=False, use_tc_tiling_on_sc=False)


def _gather_users(users, user_table):
    @functools.partial(
        pl.kernel,
        mesh=plsc.VectorSubcoreMesh(**_MESH),
        out_type=jax.ShapeDtypeStruct((_B, _D), jnp.float32),
        compiler_params=_PARAMS,
        scratch_types=[
            pltpu.VMEM((_NCHUNK, _CHUNK), jnp.int32),
            pltpu.VMEM((_RPW, _D), jnp.float32),
            pltpu.SemaphoreType.DMA,
        ],
    )
    def k(users_h, ut_h, out_h, uidx, urows, sem):
        wid = lax.axis_index("s") * _NC + lax.axis_index("c")
        base = wid * _RPW
        for j in range(_NCHUNK):
            pltpu.sync_copy(users_h.at[pl.ds(base + j * _CHUNK, _CHUNK)],
                            uidx.at[j])
        copies = []
        for j in range(_NCHUNK):
            copies.append(pltpu.async_copy(
                ut_h.at[uidx.at[j]], urows.at[pl.ds(j * _CHUNK, _CHUNK)], sem))
        for c in copies:
            c.wait()
        pltpu.sync_copy(urows, out_h.at[pl.ds(base, _RPW)])

    return k(users, user_table)


def _item_dot(items, scores, sample_weight, item_table, urows_g):
    @functools.partial(
        pl.kernel,
        mesh=plsc.VectorSubcoreMesh(**_MESH),
        out_type=jax.ShapeDtypeStruct((_NW, _L), jnp.float32),
        compiler_params=_PARAMS,
        scratch_types=[
            pltpu.VMEM((_NCHUNK, _CHUNK), jnp.int32),
            pltpu.VMEM((_RPW,), jnp.float32),
            pltpu.VMEM((_RPW,), jnp.float32),
            pltpu.VMEM((_RPW, _D), jnp.float32),
            pltpu.VMEM((_RPW, _D), jnp.float32),
            pltpu.VMEM((_L,), jnp.float32),
            pltpu.SemaphoreType.DMA,
        ],
    )
    def k(items_h, scores_h, w_h, it_h, ug_h, out_h,
          iidx, sc_v, w_v, urows, irows, ostage, sem):
        wid = lax.axis_index("s") * _NC + lax.axis_index("c")
        base = wid * _RPW
        for j in range(_NCHUNK):
            pltpu.sync_copy(items_h.at[pl.ds(base + j * _CHUNK, _CHUNK)],
                            iidx.at[j])
        pltpu.sync_copy(scores_h.at[pl.ds(base, _RPW)], sc_v)
        pltpu.sync_copy(w_h.at[pl.ds(base, _RPW)], w_v)

        copies = [pltpu.async_copy(ug_h.at[pl.ds(base, _RPW)], urows, sem)]
        for j in range(_NCHUNK):
            copies.append(pltpu.async_copy(
                it_h.at[iidx.at[j]], irows.at[pl.ds(j * _CHUNK, _CHUNK)], sem))
        for c in copies:
            c.wait()

        def group(g, acc):
            r0 = g * _L
            svec = sc_v[pl.ds(r0, _L)]
            wvec = w_v[pl.ds(r0, _L)]
            for j in range(_L):
                r = r0 + j
                p = urows[r, pl.ds(0, _L)] * irows[r, pl.ds(0, _L)]
                for c in range(1, _D // _L):
                    p = p + (urows[r, pl.ds(c * _L, _L)]
                             * irows[r, pl.ds(c * _L, _L)])
                pred = jnp.sum(p)
                e = pred - svec[j]
                acc = acc + e * e * wvec[j]
            return acc

        total = lax.fori_loop(0, _NG, group, jnp.float32(0.0))
        ostage[...] = jnp.where(lax.iota(jnp.int32, _L) == 0, total, 0.0)
        pltpu.sync_copy(ostage, out_h.at[wid])

    return k(items, scores, sample_weight, item_table, urows_g)


def _finalize(partials):
    def body(p_ref, o_ref):
        o_ref[0, 0] = jnp.sum(p_ref[...]) * (1.0 / _B)

    out = pl.pallas_call(
        body,
        out_shape=jax.ShapeDtypeStruct((1, 1), jnp.float32),
        out_specs=pl.BlockSpec(memory_space=pltpu.SMEM),
    )(partials)
    return out[0, 0]


def kernel(users, items, scores, sample_weight, user_table, item_table):
    urows_g = _gather_users(users, user_table)
    partials = _item_dot(items, scores, sample_weight, item_table, urows_g)
    return _finalize(partials)


# speedup vs baseline: 1.4874x; 1.1367x over previous
import functools

import jax
import jax.numpy as jnp
from jax import lax
from jax.experimental import pallas as pl
from jax.experimental.pallas import tpu as pltpu
from jax.experimental.pallas import tpu_sc as plsc

_B = 16384
_D = 64
_V = 1000000
_NC = 2
_NS = 16
_L = 16
_NW = _NC * _NS
_RPW = _B // _NW
_NG = _RPW // _L

_NFULL = _V // 128
_BPW = _NFULL // _NW
_NEXTRA = _NFULL - _BPW * _NW
_SPAN = _BPW * 128
_XBASE = _SPAN
_TBASE = _SPAN + 128
_MAPN = _TBASE + 64
_TAILLO = _NFULL * 128
_GN = _B + 16
_DUMP = _B

_MESH = dict(core_axis_name="c", subcore_axis_name="s")


def _iota():
    return lax.iota(jnp.int32, _L)


def _gather_table(indices, table):
    t3 = jnp.reshape(table.T, (8, 8, _V))
    tail = jnp.pad(lax.slice(table, (_TAILLO, 0), (_V, _D)),
                   ((0, 0), (0, 128 - _D)))

    @functools.partial(
        pl.kernel,
        mesh=plsc.VectorSubcoreMesh(**_MESH),
        out_type=jax.ShapeDtypeStruct((_GN, 128), jnp.float32),
        compiler_params=pltpu.CompilerParams(
            needs_layout_passes=False, use_tc_tiling_on_sc=True),
        scratch_types=[
            pltpu.VMEM((_B,), jnp.int32),
            pltpu.VMEM((_MAPN,), jnp.int32),
            pltpu.VMEM((_B,), jnp.int32),
            pltpu.VMEM((8, 8, 128), jnp.float32),
            pltpu.VMEM((8, 8, 128), jnp.float32),
            pltpu.VMEM((64, 128), jnp.float32),
            pltpu.VMEM((128, 128), jnp.float32),
            pltpu.VMEM((128,), jnp.int32),
            pltpu.VMEM((_L,), jnp.int32),
            pltpu.VMEM((_L,), jnp.int32),
            pltpu.SMEM((8,), jnp.int32),
            pltpu.SemaphoreType.DMA,
            pltpu.SemaphoreType.DMA,
            pltpu.SemaphoreType.DMA,
        ],
    )
    def k(idx_h, t3_h, tail_h, out_h,
          idxb, rowmap, ovf, bufa, bufb, tailv, extb, posl,
          tmpp, tmpr, cnts, sema, semb, semf):
        wid = lax.axis_index("s") * _NC + lax.axis_index("c")
        start = wid * _BPW
        lo = wid * _SPAN
        iv = _iota()

        pltpu.sync_copy(idx_h, idxb)
        pltpu.sync_copy(tail_h, tailv)

        def init_map(i, _):
            plsc.store_scatter(rowmap, [i * _L + iv], jnp.zeros((_L,), jnp.int32))
            return 0
        lax.fori_loop(0, _MAPN // _L, init_map, 0)
        for q in range(8):
            plsc.store_scatter(posl, [q * _L + iv],
                               jnp.full((_L,), _DUMP, jnp.int32))
        cnts[0] = 0
        cnts[1] = 0

        xlo = (_BPW * _NW + wid) * 128

        def scan(c, _):
            idxv = plsc.load_gather(idxb, [c * _L + iv])
            pos = c * _L + iv
            local0 = idxv - lo
            m0 = (local0 >= 0) & (local0 < _SPAN)
            mx = (idxv >= xlo) & (idxv < xlo + 128) & (wid < _NEXTRA)
            mt = (idxv >= _TAILLO) & (wid == _NW - 1)
            local = jnp.where(mx, idxv - xlo + _XBASE,
                              jnp.where(mt, idxv - _TAILLO + _TBASE, local0))
            m = m0 | mx | mt
            lcl = jnp.where(m, local, 0)
            p1 = pos + 1
            g0 = plsc.load_gather(rowmap, [lcl], mask=m)
            me = m & (g0 == 0)
            plsc.store_scatter(rowmap, [lcl], p1, mask=me)
            g1 = plsc.load_gather(rowmap, [lcl], mask=m)
            lost = m & (g1 != p1)
            li = jnp.where(lost, 1, 0).astype(jnp.int32)
            pref = plsc.cumsum(li)
            ov = cnts[1]
            plsc.store_scatter(ovf, [ov + pref - 1],
                               (pos << 17) | lcl, mask=lost)
            cnts[1] = ov + plsc.all_reduce_population_count(lost)[0]
            return 0
        lax.fori_loop(0, _B // _L, scan, 0)

        def flush():
            pltpu.async_copy(extb, out_h.at[posl], semf).wait()
            for q in range(8):
                plsc.store_scatter(posl, [q * _L + iv],
                                   jnp.full((_L,), _DUMP, jnp.int32))
            cnts[0] = 0

        def write_row(p1v, rlv, src, from_tail):
            cnt = cnts[0]
            cv = jnp.zeros((_L,), jnp.int32) + cnt
            for c4 in range(4):
                dl = c4 * _L + iv
                if from_tail:
                    val = plsc.load_gather(src, [rlv, dl])
                else:
                    val = plsc.load_gather(src, [dl >> 3, dl & 7, rlv])
                plsc.store_scatter(extb, [cv, dl], val)
            plsc.store_scatter(posl, [cv], p1v - 1, mask=iv == 0)
            cnts[0] = cnt + 1
            @pl.when(cnt + 1 == 128)
            def _():
                flush()

        def hits(mapv, m, rowsv, src, from_tail):
            pc = plsc.all_reduce_population_count(m)[0]
            @pl.when(pc > 0)
            def _():
                li = jnp.where(m, 1, 0).astype(jnp.int32)
                pref = plsc.cumsum(li)
                plsc.store_scatter(tmpp, [pref - 1], mapv, mask=m)
                plsc.store_scatter(tmpr, [pref - 1], rowsv, mask=m)

                def hit(kk, _):
                    kv = jnp.zeros((_L,), jnp.int32) + kk
                    p1v = plsc.load_gather(tmpp, [kv])
                    rlv = plsc.load_gather(tmpr, [kv])
                    write_row(p1v, rlv, src, from_tail)
                    return 0
                lax.fori_loop(0, pc, hit, 0)

        def process(lbase, nrows, src, from_tail):
            for cc in range(nrows // _L):
                mapv = plsc.load_gather(rowmap, [lbase + cc * _L + iv])
                hits(mapv, mapv != 0, cc * _L + iv, src, from_tail)
            nov = cnts[1]

            def ovblk(kk, _):
                lanes = kk * _L + iv
                ovv = plsc.load_gather(ovf, [lanes])
                valid = lanes < nov
                lcl = ovv & 0x1FFFF
                p1 = (ovv >> 17) + 1
                m = valid & (lcl >= lbase) & (lcl < lbase + nrows)
                hits(jnp.where(m, p1, 0), m, lcl - lbase, src, from_tail)
                return 0
            lax.fori_loop(0, (nov + _L - 1) // _L, ovblk, 0)

        def blkstep(bl, buf, sem):
            pltpu.make_async_copy(
                t3_h.at[:, :, pl.ds(0, 128)], buf, sem).wait()
            process(bl * 128, 128, buf, False)
            nb = bl + 2
            @pl.when(nb < _BPW)
            def _():
                pltpu.async_copy(
                    t3_h.at[:, :, pl.ds((start + nb) * 128, 128)], buf, sem)

        pltpu.async_copy(t3_h.at[:, :, pl.ds(start * 128, 128)], bufa, sema)
        pltpu.async_copy(t3_h.at[:, :, pl.ds((start + 1) * 128, 128)], bufb, semb)

        def pair(q, _):
            blkstep(2 * q, bufa, sema)
            blkstep(2 * q + 1, bufb, semb)
            return 0
        lax.fori_loop(0, _BPW // 2, pair, 0)

        @pl.when(wid < _NEXTRA)
        def _():
            pltpu.async_copy(
                t3_h.at[:, :, pl.ds((_BPW * _NW + wid) * 128, 128)], bufa, sema)
            pltpu.make_async_copy(
                t3_h.at[:, :, pl.ds(0, 128)], bufa, sema).wait()
            process(_XBASE, 128, bufa, False)

        @pl.when(wid == _NW - 1)
        def _():
            process(_TBASE, 64, tailv, True)

        flush()

    return k(indices, t3, tail)


def _dot_partials(scores, sample_weight, ug, ig):
    @functools.partial(
        pl.kernel,
        mesh=plsc.VectorSubcoreMesh(**_MESH),
        out_type=jax.ShapeDtypeStruct((_NW, _L), jnp.float32),
        compiler_params=pltpu.CompilerParams(
            needs_layout_passes=False, use_tc_tiling_on_sc=False),
        scratch_types=[
            pltpu.VMEM((_RPW,), jnp.float32),
            pltpu.VMEM((_RPW,), jnp.float32),
            pltpu.VMEM((128, 128), jnp.float32),
            pltpu.VMEM((128, 128), jnp.float32),
            pltpu.VMEM((_L,), jnp.float32),
            pltpu.SemaphoreType.DMA,
        ],
    )
    def k(scores_h, w_h, ug_h, ig_h, out_h, sc_v, w_v, ub, ib, ostage, sem):
        wid = lax.axis_index("s") * _NC + lax.axis_index("c")
        base = wid * _RPW
        pltpu.sync_copy(scores_h.at[pl.ds(base, _RPW)], sc_v)
        pltpu.sync_copy(w_h.at[pl.ds(base, _RPW)], w_v)

        acc = jnp.zeros((_L,), jnp.float32)
        total = jnp.float32(0.0)

        def chunk(q, total):
            c0 = base + q * 128
            cu = pltpu.async_copy(ug_h.at[pl.ds(c0, 128)], ub, sem)
            ci = pltpu.async_copy(ig_h.at[pl.ds(c0, 128)], ib, sem)
            cu.wait()
            ci.wait()

            def group(g, t):
                r0 = g * _L
                svec = sc_v[pl.ds(q * 128 + r0, _L)]
                wvec = w_v[pl.ds(q * 128 + r0, _L)]
                for j in range(_L):
                    r = r0 + j
                    p = ub[r, pl.ds(0, _L)] * ib[r, pl.ds(0, _L)]
                    for c in range(1, _D // _L):
                        p = p + (ub[r, pl.ds(c * _L, _L)]
                                 * ib[r, pl.ds(c * _L, _L)])
                    pred = jnp.sum(p)
                    e = pred - svec[j]
                    t = t + e * e * wvec[j]
                return t
            return lax.fori_loop(0, 128 // _L, group, total)

        total = lax.fori_loop(0, _RPW // 128, chunk, total)
        ostage[...] = jnp.where(_iota() == 0, total, 0.0)
        pltpu.sync_copy(ostage, out_h.at[wid])

    return k(scores, sample_weight, ug, ig)


def _finalize(partials):
    def body(p_ref, o_ref):
        o_ref[0, 0] = jnp.sum(p_ref[...]) * (1.0 / _B)

    out = pl.pallas_call(
        body,
        out_shape=jax.ShapeDtypeStruct((1, 1), jnp.float32),
        out_specs=pl.BlockSpec(memory_space=pltpu.SMEM),
    )(partials)
    return out[0, 0]


def kernel(users, items, scores, sample_weight, user_table, item_table):
    ug = _gather_table(users, user_table)
    ig = _gather_table(items, item_table)
    partials = _dot_partials(scores, sample_weight, ug, ig)
    return _finalize(partials)
